# Initial kernel scaffold; baseline (speedup 1.0000x reference)
import jax
import jax.numpy as jnp
from jax.experimental import pallas as pl


def kernel(h, batch_idx, W_node, b_node, W_act, b_act):
    raise NotImplementedError("write your pallas kernel here")


# speedup vs baseline: 12.6410x; 12.6410x over previous
import functools
import jax
import jax.numpy as jnp
from jax import lax
from jax.experimental import pallas as pl
from jax.experimental.pallas import tpu as pltpu
from jax.experimental.pallas import tpu_sc as plsc

N_ACT = 128
G = 1024
F = 16
NWORK = 32
RPW = 3136
NPAD = NWORK * RPW
WSIZE = 8192
NPAD2 = NPAD + WSIZE
OFFW = 1040
BL = 2048
NEG = -1e9


def _tc1_body(n_real, hb, g1b, wn, bn, nl_out, sc_out):
    j = pl.program_id(0)
    h = hb[0]
    nlc = jax.lax.dot_general(
        h, wn[...], (((1,), (0,)), ((), ())),
        preferred_element_type=jnp.float32)
    nl = nlc.reshape(1, BL) + bn[0, 0]
    lanes = jax.lax.broadcasted_iota(jnp.int32, (1, BL), 1) + j * BL
    valid = lanes < n_real
    nl_out[0] = jnp.where(valid, nl, NEG)
    sc_out[0] = jnp.where(valid, nl + g1b[0], NEG)


def _tc1(h3, g12, wn, bn, n_real):
    nb = NPAD2 // BL
    return pl.pallas_call(
        functools.partial(_tc1_body, n_real),
        grid=(nb,),
        in_specs=[
            pl.BlockSpec((1, BL, F), lambda j: (j, 0, 0)),
            pl.BlockSpec((1, 1, BL), lambda j: (j, 0, 0)),
            pl.BlockSpec((F, 1), lambda j: (0, 0)),
            pl.BlockSpec((1, 1), lambda j: (0, 0)),
        ],
        out_specs=[
            pl.BlockSpec((1, 1, BL), lambda j: (j, 0, 0)),
            pl.BlockSpec((1, 1, BL), lambda j: (j, 0, 0)),
        ],
        out_shape=[
            jax.ShapeDtypeStruct((nb, 1, BL), jnp.float32),
            jax.ShapeDtypeStruct((nb, 1, BL), jnp.float32),
        ],
    )(h3, g12, wn, bn)


def _sca_body(n_real, bidx_hbm, offtab_hbm, bbuf, offloc, dsem):
    widx = lax.axis_index("s") * 2 + lax.axis_index("c")
    pltpu.sync_copy(bidx_hbm.at[pl.ds(widx * RPW, RPW + 16)], bbuf)
    iota = lax.iota(jnp.int32, 16)
    neg1 = jnp.full((16,), -1, jnp.int32)
    for i in range(OFFW // 16):
        offloc[pl.ds(i * 16, 16)] = neg1
    offloc[pl.ds(G, 16)] = jnp.where(iota == 0, n_real, -1)

    def step(c, _):
        cur = bbuf[pl.ds(16 + c * 16, 16)]
        prev = bbuf[pl.ds(15 + c * 16, 16)]
        bmask = cur != prev
        posv = widx * RPW + c * 16 + iota
        plsc.store_scatter(offloc, [cur], posv, mask=bmask)
        return _

    lax.fori_loop(0, RPW // 16, step, 0)
    pltpu.sync_copy(offloc, offtab_hbm.at[pl.ds(widx * OFFW, OFFW)])


def _sca(bidx_p, n_real):
    mesh = plsc.VectorSubcoreMesh(core_axis_name="c", subcore_axis_name="s")
    kern = pl.kernel(
        functools.partial(_sca_body, n_real),
        out_type=jax.ShapeDtypeStruct((NWORK * OFFW,), jnp.int32),
        mesh=mesh,
        scratch_types=[
            pltpu.VMEM((RPW + 16,), jnp.int32),
            pltpu.VMEM((OFFW,), jnp.int32),
            pltpu.SemaphoreType.DMA,
        ],
        compiler_params=pltpu.CompilerParams(needs_layout_passes=False),
    )
    return kern(bidx_p)


def _scb_body(nl_hbm, sc_hbm, h_hbm, offtab_hbm,
              m_hbm, s_hbm, d_hbm, a_hbm, nla_hbm, hsel_hbm,
              nlbuf, scbuf, offrows, offmer,
              outm, outs, outd, outnla, outa, idxe, hselloc, dsem):
    widx = lax.axis_index("s") * 2 + lax.axis_index("c")
    iota = lax.iota(jnp.int32, 16)
    sb = widx * 32

    pltpu.sync_copy(offtab_hbm, offrows)
    for kk in range(3):
        acc = jnp.full((16,), -1, jnp.int32)
        for r in range(NWORK):
            acc = jnp.maximum(acc, offrows[pl.ds(r * OFFW + sb + kk * 16, 16)])
        offmer[pl.ds(kk * 16, 16)] = acc

    def put(ref, j, val):
        plsc.store_scatter(ref, [jnp.full((16,), j, jnp.int32)],
                           jnp.full((16,), val), mask=(iota == 0))

    def seg_step(j, cur_w):
        vv = offmer[pl.ds(j, 16)]
        s0 = jnp.maximum(vv[0], 0)
        e0 = jnp.maximum(vv[1], s0)

        def ensure(pos, cw):
            need = (pos < cw) | (pos + 16 > cw + WSIZE)
            nw = pl.multiple_of(jnp.where(need, pos - lax.rem(pos, 8), cw), 8)

            @pl.when(need)
            def _():
                pltpu.sync_copy(nl_hbm.at[pl.ds(nw, WSIZE)], nlbuf)
                pltpu.sync_copy(sc_hbm.at[pl.ds(nw, WSIZE)], scbuf)

            return nw

        def bodyA(carry):
            pos, cw, mvec, bvec, ivec, nlvec = carry
            cw = ensure(pos, cw)
            off = pos - cw
            v = nlbuf[pl.ds(off, 16)]
            sv = scbuf[pl.ds(off, 16)]
            lm = iota < (e0 - pos)
            vm = jnp.where(lm, v, NEG)
            svm = jnp.where(lm, sv, NEG)
            mvec = jnp.maximum(mvec, vm)
            take = svm >= bvec
            bvec = jnp.where(take, svm, bvec)
            ivec = jnp.where(take, pos + iota, ivec)
            nlvec = jnp.where(take, vm, nlvec)
            return pos + 16, cw, mvec, bvec, ivec, nlvec

        init = (s0, cur_w,
                jnp.full((16,), NEG), jnp.full((16,), NEG),
                jnp.full((16,), -1, jnp.int32), jnp.full((16,), NEG))
        pos, cur_w, mvec, bvec, ivec, nlvec = lax.while_loop(
            lambda c: c[0] < e0, bodyA, init)

        m = jnp.max(mvec)
        b2 = jnp.max(bvec)
        a = jnp.max(jnp.where(bvec == b2, ivec, -1))
        nla = jnp.max(jnp.where(ivec == a, nlvec, NEG))
        a = jnp.maximum(a, 0)

        def bodyB(carry):
            pos, cw, svec, dvec = carry
            cw = ensure(pos, cw)
            off = pos - cw
            v = nlbuf[pl.ds(off, 16)]
            lm = iota < (e0 - pos)
            sh = v - m
            ex = jnp.where(lm, jnp.exp(sh), jnp.float32(0.0))
            return pos + 16, cw, svec + ex, dvec + ex * sh

        initb = (s0, cur_w, jnp.zeros((16,), jnp.float32),
                 jnp.zeros((16,), jnp.float32))
        pos, cur_w, svec, dvec = lax.while_loop(
            lambda c: c[0] < e0, bodyB, initb)

        put(outm, j, m)
        put(outs, j, jnp.sum(svec))
        put(outd, j, jnp.sum(dvec))
        put(outnla, j, nla)
        put(outa, j, a)
        return cur_w

    lax.fori_loop(0, 32, seg_step, jnp.int32(-2 ** 30))

    pltpu.sync_copy(outm, m_hbm.at[pl.ds(sb, 32)])
    pltpu.sync_copy(outs, s_hbm.at[pl.ds(sb, 32)])
    pltpu.sync_copy(outd, d_hbm.at[pl.ds(sb, 32)])
    pltpu.sync_copy(outnla, nla_hbm.at[pl.ds(sb, 32)])
    pltpu.sync_copy(outa, a_hbm.at[pl.ds(sb, 32)])
    va = outa[pl.ds(0, 16)]
    vb = outa[pl.ds(16, 16)]
    for k in range(32):
        ak = va[k] if k < 16 else vb[k - 16]
        idxe[k // 8, pl.ds((k % 8) * 16, 16)] = ak * F + iota
    for j in range(4):
        pltpu.async_copy(h_hbm.at[idxe.at[j]], hselloc.at[j], dsem).wait()
        pltpu.sync_copy(hselloc.at[j],
                        hsel_hbm.at[pl.ds(sb * F + j * 128, 128)])


def _scb(nl, sc, h2d, offtabs):
    mesh = plsc.VectorSubcoreMesh(core_axis_name="c", subcore_axis_name="s")
    kern = pl.kernel(
        _scb_body,
        out_type=(
            jax.ShapeDtypeStruct((G,), jnp.float32),
            jax.ShapeDtypeStruct((G,), jnp.float32),
            jax.ShapeDtypeStruct((G,), jnp.float32),
            jax.ShapeDtypeStruct((G,), jnp.int32),
            jax.ShapeDtypeStruct((G,), jnp.float32),
            jax.ShapeDtypeStruct((G * F,), jnp.float32),
        ),
        mesh=mesh,
        scratch_types=[
            pltpu.VMEM((WSIZE,), jnp.float32),
            pltpu.VMEM((WSIZE,), jnp.float32),
            pltpu.VMEM((NWORK * OFFW,), jnp.int32),
            pltpu.VMEM((48,), jnp.int32),
            pltpu.VMEM((32,), jnp.float32),
            pltpu.VMEM((32,), jnp.float32),
            pltpu.VMEM((32,), jnp.float32),
            pltpu.VMEM((32,), jnp.float32),
            pltpu.VMEM((32,), jnp.int32),
            pltpu.VMEM((4, 128), jnp.int32),
            pltpu.VMEM((4, 128), jnp.float32),
            pltpu.SemaphoreType.DMA,
        ],
        compiler_params=pltpu.CompilerParams(needs_layout_passes=False),
    )
    return kern(nl, sc, h2d, offtabs)


def _tc2_body(hs, wa, ba, g2, mv, sv, dv, nla, lp_out, ent_out, act_out):
    al = jax.lax.dot_general(
        hs[...], wa[...], (((1,), (0,)), ((), ())),
        preferred_element_type=jnp.float32) + ba[...]
    rmax = jnp.max(al, axis=1, keepdims=True)
    sha = al - rmax
    lse = jnp.log(jnp.sum(jnp.exp(sha), axis=1, keepdims=True))
    logp = sha - lse
    sc2 = al + g2[...]
    smax = jnp.max(sc2, axis=1, keepdims=True)
    lanes = jax.lax.broadcasted_iota(jnp.int32, (G, N_ACT), 1)
    aidx = jnp.min(jnp.where(sc2 == smax, lanes, N_ACT), axis=1,
                   keepdims=True)
    alogp = jnp.sum(jnp.where(lanes == aidx, logp, 0.0), axis=1,
                    keepdims=True)
    aent = -jnp.sum(jnp.exp(logp) * logp, axis=1, keepdims=True)

    S = sv[...]
    logS = jnp.log(S)
    lp = (nla[...] - mv[...] - logS) + alogp.reshape(8, N_ACT)
    ent = (logS - dv[...] / S) + aent.reshape(8, N_ACT)
    lp_out[...] = lp
    ent_out[...] = ent
    act_out[...] = aidx.reshape(8, N_ACT)


def _tc2(hsel, wa, ba, g2, mv, sv, dv, nla):
    return pl.pallas_call(
        _tc2_body,
        out_shape=[
            jax.ShapeDtypeStruct((8, N_ACT), jnp.float32),
            jax.ShapeDtypeStruct((8, N_ACT), jnp.float32),
            jax.ShapeDtypeStruct((8, N_ACT), jnp.int32),
        ],
    )(hsel, wa, ba, g2, mv, sv, dv, nla)


def kernel(h, batch_idx, W_node, b_node, W_act, b_act):
    n = h.shape[0]

    kg = jax.random.key(42)
    g1 = jax.random.gumbel(kg, (n,), dtype=jnp.float32)
    g2 = jax.random.gumbel(jax.random.fold_in(kg, 1), (G, N_ACT),
                           dtype=jnp.float32)

    h_p = jnp.pad(h.astype(jnp.float32), ((0, NPAD2 - n), (0, 0)))
    h3 = h_p.reshape(NPAD2 // BL, BL, F)
    g1p = jnp.pad(g1, (0, NPAD2 - n)).reshape(NPAD2 // BL, 1, BL)
    bidx_p = jnp.concatenate([
        jnp.full((16,), -1, jnp.int32),
        batch_idx.astype(jnp.int32),
        jnp.full((NPAD - n,), G - 1, jnp.int32),
    ])

    nl2, sc2 = _tc1(h3, g1p, W_node.astype(jnp.float32),
                    b_node.reshape(1, 1).astype(jnp.float32), n)
    nl = nl2.reshape(NPAD2)
    sc = sc2.reshape(NPAD2)

    offtabs = _sca(bidx_p, n)
    mv, sv, dv, nidx, nla, hselflat = _scb(nl, sc, h_p.reshape(NPAD2 * F),
                                           offtabs)

    lp, ent, act = _tc2(
        hselflat.reshape(G, F), W_act.astype(jnp.float32),
        b_act.reshape(1, N_ACT).astype(jnp.float32),
        g2, mv.reshape(8, N_ACT), sv.reshape(8, N_ACT),
        dv.reshape(8, N_ACT), nla.reshape(8, N_ACT))

    actions = jnp.stack([nidx, act.reshape(G)], axis=-1)
    return actions, lp.reshape(G), ent.reshape(G)


# speedup vs baseline: 13.7909x; 1.0910x over previous
import functools
import jax
import jax.numpy as jnp
from jax import lax
from jax.experimental import pallas as pl
from jax.experimental.pallas import tpu as pltpu
from jax.experimental.pallas import tpu_sc as plsc

N_ACT = 128
G = 1024
F = 16
NWORK = 32
RPW = 3136
NPAD = NWORK * RPW
WSIZE = 8192
NPAD2 = NPAD + WSIZE
OFFW = 1040
BL = 2048
NEG = -1e9


RROW = 500
NROWS = 12500


def _tc1_body(hb, g1b, mm, bn, nl_out, sc_out):
    pk = jax.lax.dot_general(
        hb[0], mm[...], (((1,), (0,)), ((), ())),
        preferred_element_type=jnp.float32) + bn[0, 0]
    nl_out[0] = pk
    sc_out[0] = pk + g1b[0]


def _tc1(h3, g13, mm, bn):
    nb = NROWS // RROW
    return pl.pallas_call(
        _tc1_body,
        grid=(nb,),
        in_specs=[
            pl.BlockSpec((1, RROW, 128), lambda j: (j, 0, 0)),
            pl.BlockSpec((1, RROW, 8), lambda j: (j, 0, 0)),
            pl.BlockSpec((128, 8), lambda j: (0, 0)),
            pl.BlockSpec((1, 1), lambda j: (0, 0)),
        ],
        out_specs=[
            pl.BlockSpec((1, RROW, 8), lambda j: (j, 0, 0)),
            pl.BlockSpec((1, RROW, 8), lambda j: (j, 0, 0)),
        ],
        out_shape=[
            jax.ShapeDtypeStruct((nb, RROW, 8), jnp.float32),
            jax.ShapeDtypeStruct((nb, RROW, 8), jnp.float32),
        ],
    )(h3, g13, mm, bn)


def _sca_body(n_real, bidx_hbm, offtab_hbm, bbuf, offloc, dsem):
    widx = lax.axis_index("s") * 2 + lax.axis_index("c")
    pltpu.sync_copy(bidx_hbm.at[pl.ds(widx * RPW, RPW + 16)], bbuf)
    iota = lax.iota(jnp.int32, 16)
    neg1 = jnp.full((16,), -1, jnp.int32)
    for i in range(OFFW // 16):
        offloc[pl.ds(i * 16, 16)] = neg1
    offloc[pl.ds(G, 16)] = jnp.where(iota == 0, n_real, -1)

    def step(c, _):
        cur = bbuf[pl.ds(16 + c * 16, 16)]
        prev = bbuf[pl.ds(15 + c * 16, 16)]
        bmask = cur != prev
        posv = widx * RPW + c * 16 + iota
        plsc.store_scatter(offloc, [cur], posv, mask=bmask)
        return _

    lax.fori_loop(0, RPW // 16, step, 0)
    pltpu.sync_copy(offloc, offtab_hbm.at[pl.ds(widx * OFFW, OFFW)])


def _sca(bidx_p, n_real):
    mesh = plsc.VectorSubcoreMesh(core_axis_name="c", subcore_axis_name="s")
    kern = pl.kernel(
        functools.partial(_sca_body, n_real),
        out_type=jax.ShapeDtypeStruct((NWORK * OFFW,), jnp.int32),
        mesh=mesh,
        scratch_types=[
            pltpu.VMEM((RPW + 16,), jnp.int32),
            pltpu.VMEM((OFFW,), jnp.int32),
            pltpu.SemaphoreType.DMA,
        ],
        compiler_params=pltpu.CompilerParams(needs_layout_passes=False),
    )
    return kern(bidx_p)


def _scb_body(nl_hbm, sc_hbm, h_hbm, offtab_hbm,
              m_hbm, s_hbm, d_hbm, a_hbm, nla_hbm, hsel_hbm,
              nlbuf, scbuf, offrows, offmer,
              outm, outs, outd, outnla, outa, idxe, hselloc, dsem):
    widx = lax.axis_index("s") * 2 + lax.axis_index("c")
    iota = lax.iota(jnp.int32, 16)
    sb = widx * 32

    pltpu.sync_copy(offtab_hbm, offrows)
    for kk in range(3):
        acc = jnp.full((16,), -1, jnp.int32)
        for r in range(NWORK):
            acc = jnp.maximum(acc, offrows[pl.ds(r * OFFW + sb + kk * 16, 16)])
        offmer[pl.ds(kk * 16, 16)] = acc

    def put(ref, j, val):
        plsc.store_scatter(ref, [jnp.full((16,), j, jnp.int32)],
                           jnp.full((16,), val), mask=(iota == 0))

    def seg_step(j, cur_w):
        vv = offmer[pl.ds(j, 16)]
        s0 = jnp.maximum(vv[0], 0)
        e0 = jnp.maximum(vv[1], s0)

        def ensure(pos, cw):
            need = (pos < cw) | (pos + 16 > cw + WSIZE)
            nw = pl.multiple_of(jnp.where(need, pos - lax.rem(pos, 8), cw), 8)

            @pl.when(need)
            def _():
                pltpu.sync_copy(nl_hbm.at[pl.ds(nw, WSIZE)], nlbuf)
                pltpu.sync_copy(sc_hbm.at[pl.ds(nw, WSIZE)], scbuf)

            return nw

        def bodyA(carry):
            pos, cw, mvec, bvec, ivec, nlvec = carry
            cw = ensure(pos, cw)
            off = pos - cw
            v = nlbuf[pl.ds(off, 16)]
            sv = scbuf[pl.ds(off, 16)]
            lm = iota < (e0 - pos)
            vm = jnp.where(lm, v, NEG)
            svm = jnp.where(lm, sv, NEG)
            mvec = jnp.maximum(mvec, vm)
            take = svm >= bvec
            bvec = jnp.where(take, svm, bvec)
            ivec = jnp.where(take, pos + iota, ivec)
            nlvec = jnp.where(take, vm, nlvec)
            return pos + 16, cw, mvec, bvec, ivec, nlvec

        init = (s0, cur_w,
                jnp.full((16,), NEG), jnp.full((16,), NEG),
                jnp.full((16,), -1, jnp.int32), jnp.full((16,), NEG))
        pos, cur_w, mvec, bvec, ivec, nlvec = lax.while_loop(
            lambda c: c[0] < e0, bodyA, init)

        m = jnp.max(mvec)
        b2 = jnp.max(bvec)
        a = jnp.max(jnp.where(bvec == b2, ivec, -1))
        nla = jnp.max(jnp.where(ivec == a, nlvec, NEG))
        a = jnp.maximum(a, 0)

        def bodyB(carry):
            pos, cw, svec, dvec = carry
            cw = ensure(pos, cw)
            off = pos - cw
            v = nlbuf[pl.ds(off, 16)]
            lm = iota < (e0 - pos)
            sh = v - m
            ex = jnp.where(lm, jnp.exp(sh), jnp.float32(0.0))
            return pos + 16, cw, svec + ex, dvec + ex * sh

        initb = (s0, cur_w, jnp.zeros((16,), jnp.float32),
                 jnp.zeros((16,), jnp.float32))
        pos, cur_w, svec, dvec = lax.while_loop(
            lambda c: c[0] < e0, bodyB, initb)

        put(outm, j, m)
        put(outs, j, jnp.sum(svec))
        put(outd, j, jnp.sum(dvec))
        put(outnla, j, nla)
        put(outa, j, a)
        return cur_w

    lax.fori_loop(0, 32, seg_step, jnp.int32(-2 ** 30))

    pltpu.sync_copy(outm, m_hbm.at[pl.ds(sb, 32)])
    pltpu.sync_copy(outs, s_hbm.at[pl.ds(sb, 32)])
    pltpu.sync_copy(outd, d_hbm.at[pl.ds(sb, 32)])
    pltpu.sync_copy(outnla, nla_hbm.at[pl.ds(sb, 32)])
    pltpu.sync_copy(outa, a_hbm.at[pl.ds(sb, 32)])
    va = outa[pl.ds(0, 16)]
    vb = outa[pl.ds(16, 16)]
    for k in range(32):
        ak = va[k] if k < 16 else vb[k - 16]
        idxe[k // 8, pl.ds((k % 8) * 16, 16)] = ak * F + iota
    for j in range(4):
        pltpu.async_copy(h_hbm.at[idxe.at[j]], hselloc.at[j], dsem).wait()
        pltpu.sync_copy(hselloc.at[j],
                        hsel_hbm.at[pl.ds(sb * F + j * 128, 128)])


def _scb(nl, sc, h2d, offtabs):
    mesh = plsc.VectorSubcoreMesh(core_axis_name="c", subcore_axis_name="s")
    kern = pl.kernel(
        _scb_body,
        out_type=(
            jax.ShapeDtypeStruct((G,), jnp.float32),
            jax.ShapeDtypeStruct((G,), jnp.float32),
            jax.ShapeDtypeStruct((G,), jnp.float32),
            jax.ShapeDtypeStruct((G,), jnp.int32),
            jax.ShapeDtypeStruct((G,), jnp.float32),
            jax.ShapeDtypeStruct((G * F,), jnp.float32),
        ),
        mesh=mesh,
        scratch_types=[
            pltpu.VMEM((WSIZE,), jnp.float32),
            pltpu.VMEM((WSIZE,), jnp.float32),
            pltpu.VMEM((NWORK * OFFW,), jnp.int32),
            pltpu.VMEM((48,), jnp.int32),
            pltpu.VMEM((32,), jnp.float32),
            pltpu.VMEM((32,), jnp.float32),
            pltpu.VMEM((32,), jnp.float32),
            pltpu.VMEM((32,), jnp.float32),
            pltpu.VMEM((32,), jnp.int32),
            pltpu.VMEM((4, 128), jnp.int32),
            pltpu.VMEM((4, 128), jnp.float32),
            pltpu.SemaphoreType.DMA,
        ],
        compiler_params=pltpu.CompilerParams(needs_layout_passes=False),
    )
    return kern(nl, sc, h2d, offtabs)


def _tc2_body(hs, wa, ba, g2, mv, sv, dv, nla, lp_out, ent_out, act_out):
    al = jax.lax.dot_general(
        hs[...], wa[...], (((1,), (0,)), ((), ())),
        preferred_element_type=jnp.float32) + ba[...]
    rmax = jnp.max(al, axis=1, keepdims=True)
    sha = al - rmax
    lse = jnp.log(jnp.sum(jnp.exp(sha), axis=1, keepdims=True))
    logp = sha - lse
    sc2 = al + g2[...]
    smax = jnp.max(sc2, axis=1, keepdims=True)
    lanes = jax.lax.broadcasted_iota(jnp.int32, (G, N_ACT), 1)
    aidx = jnp.min(jnp.where(sc2 == smax, lanes, N_ACT), axis=1,
                   keepdims=True)
    alogp = jnp.sum(jnp.where(lanes == aidx, logp, 0.0), axis=1,
                    keepdims=True)
    aent = -jnp.sum(jnp.exp(logp) * logp, axis=1, keepdims=True)

    S = sv[...]
    logS = jnp.log(S)
    lp = (nla[...] - mv[...] - logS) + alogp.reshape(8, N_ACT)
    ent = (logS - dv[...] / S) + aent.reshape(8, N_ACT)
    lp_out[...] = lp
    ent_out[...] = ent
    act_out[...] = aidx.reshape(8, N_ACT)


def _tc2(hsel, wa, ba, g2, mv, sv, dv, nla):
    return pl.pallas_call(
        _tc2_body,
        out_shape=[
            jax.ShapeDtypeStruct((8, N_ACT), jnp.float32),
            jax.ShapeDtypeStruct((8, N_ACT), jnp.float32),
            jax.ShapeDtypeStruct((8, N_ACT), jnp.int32),
        ],
    )(hsel, wa, ba, g2, mv, sv, dv, nla)


def kernel(h, batch_idx, W_node, b_node, W_act, b_act):
    n = h.shape[0]

    kg = jax.random.key(42)
    g1 = jax.random.gumbel(kg, (n,), dtype=jnp.float32)
    g2 = jax.random.gumbel(jax.random.fold_in(kg, 1), (G, N_ACT),
                           dtype=jnp.float32)

    hlin = h.astype(jnp.float32).reshape(NROWS, 128)
    h3 = hlin.reshape(NROWS // RROW, RROW, 128)
    g13 = g1.reshape(NROWS // RROW, RROW, 8)
    wvec = W_node.astype(jnp.float32).reshape(1, F)
    mm = (jnp.tile(wvec, (8, 1)).reshape(128, 1)
          * jnp.repeat(jnp.eye(8, dtype=jnp.float32), F, axis=0))
    bidx_p = jnp.concatenate([
        jnp.full((16,), -1, jnp.int32),
        batch_idx.astype(jnp.int32),
        jnp.full((NPAD - n,), G - 1, jnp.int32),
    ])

    nl2, sc2 = _tc1(h3, g13, mm,
                    b_node.reshape(1, 1).astype(jnp.float32))
    nl = jnp.pad(nl2.reshape(n), (0, NPAD2 - n))
    sc = jnp.pad(sc2.reshape(n), (0, NPAD2 - n))

    offtabs = _sca(bidx_p, n)
    mv, sv, dv, nidx, nla, hselflat = _scb(nl, sc, hlin.reshape(n * F),
                                           offtabs)

    lp, ent, act = _tc2(
        hselflat.reshape(G, F), W_act.astype(jnp.float32),
        b_act.reshape(1, N_ACT).astype(jnp.float32),
        g2, mv.reshape(8, N_ACT), sv.reshape(8, N_ACT),
        dv.reshape(8, N_ACT), nla.reshape(8, N_ACT))

    actions = jnp.stack([nidx, act.reshape(G)], axis=-1)
    return actions, lp.reshape(G), ent.reshape(G)


# speedup vs baseline: 14.4559x; 1.0482x over previous
import functools
import jax
import jax.numpy as jnp
from jax import lax
from jax.experimental import pallas as pl
from jax.experimental.pallas import tpu as pltpu
from jax.experimental.pallas import tpu_sc as plsc

N_ACT = 128
G = 1024
F = 16
NWORK = 32
RPW = 3136
NPAD = NWORK * RPW
WSIZE = 8192
NPAD2 = NPAD + WSIZE
OFFW = 1040
BL = 2048
NEG = -1e9


RROW = 500
NROWS = 12500


def _tc1_body(hb, g1b, mm, bn, nl_out, sc_out):
    pk = jax.lax.dot_general(
        hb[0], mm[...], (((1,), (0,)), ((), ())),
        preferred_element_type=jnp.float32) + bn[0, 0]
    nl_out[0] = pk
    sc_out[0] = pk + g1b[0]


def _tc1(h3, g13, mm, bn):
    nb = NROWS // RROW
    return pl.pallas_call(
        _tc1_body,
        grid=(nb,),
        in_specs=[
            pl.BlockSpec((1, RROW, 128), lambda j: (j, 0, 0)),
            pl.BlockSpec((1, RROW, 8), lambda j: (j, 0, 0)),
            pl.BlockSpec((128, 8), lambda j: (0, 0)),
            pl.BlockSpec((1, 1), lambda j: (0, 0)),
        ],
        out_specs=[
            pl.BlockSpec((1, RROW, 8), lambda j: (j, 0, 0)),
            pl.BlockSpec((1, RROW, 8), lambda j: (j, 0, 0)),
        ],
        out_shape=[
            jax.ShapeDtypeStruct((nb, RROW, 8), jnp.float32),
            jax.ShapeDtypeStruct((nb, RROW, 8), jnp.float32),
        ],
    )(h3, g13, mm, bn)


def _sca_body(n_real, bidx_hbm, offtab_hbm, bbuf, offloc, dsem):
    widx = lax.axis_index("s") * 2 + lax.axis_index("c")
    iota = lax.iota(jnp.int32, 16)
    neg1 = jnp.full((16,), -1, jnp.int32)
    last_n = n_real - (NWORK - 1) * RPW

    @pl.when(widx < NWORK - 1)
    def _():
        pltpu.sync_copy(bidx_hbm.at[pl.ds(widx * RPW, RPW)],
                        bbuf.at[pl.ds(16, RPW)])

    @pl.when(widx == NWORK - 1)
    def _():
        pltpu.sync_copy(bidx_hbm.at[pl.ds((NWORK - 1) * RPW, last_n)],
                        bbuf.at[pl.ds(16, last_n)])

    @pl.when(widx > 0)
    def _():
        pltpu.sync_copy(bidx_hbm.at[pl.ds(widx * RPW - 16, 16)],
                        bbuf.at[pl.ds(0, 16)])

    @pl.when(widx == 0)
    def _():
        bbuf[pl.ds(0, 16)] = neg1

    for i in range(OFFW // 16):
        offloc[pl.ds(i * 16, 16)] = neg1
    offloc[pl.ds(G, 16)] = jnp.where(iota == 0, n_real, -1)

    def step(c, _):
        cur = bbuf[pl.ds(16 + c * 16, 16)]
        prev = bbuf[pl.ds(15 + c * 16, 16)]
        bmask = cur != prev
        posv = widx * RPW + c * 16 + iota
        plsc.store_scatter(offloc, [cur], posv, mask=bmask)
        return _

    nchunks = jnp.where(widx == NWORK - 1, last_n // 16, RPW // 16)
    lax.fori_loop(0, nchunks, step, 0)
    pltpu.sync_copy(offloc, offtab_hbm.at[pl.ds(widx * OFFW, OFFW)])


def _sca(bidx_p, n_real):
    mesh = plsc.VectorSubcoreMesh(core_axis_name="c", subcore_axis_name="s")
    kern = pl.kernel(
        functools.partial(_sca_body, n_real),
        out_type=jax.ShapeDtypeStruct((NWORK * OFFW,), jnp.int32),
        mesh=mesh,
        scratch_types=[
            pltpu.VMEM((RPW + 16,), jnp.int32),
            pltpu.VMEM((OFFW,), jnp.int32),
            pltpu.SemaphoreType.DMA,
        ],
        compiler_params=pltpu.CompilerParams(needs_layout_passes=False),
    )
    return kern(bidx_p)


def _scb_body(nl_hbm, sc_hbm, h_hbm, offtab_hbm,
              m_hbm, s_hbm, d_hbm, a_hbm, nla_hbm, hsel_hbm,
              nlbuf, scbuf, offrows, offmer,
              outm, outs, outd, outnla, outa, rowscr, rowsbuf,
              hselloc, dsem):
    widx = lax.axis_index("s") * 2 + lax.axis_index("c")
    iota = lax.iota(jnp.int32, 16)
    sb = widx * 32

    pltpu.sync_copy(offtab_hbm, offrows)
    for kk in range(3):
        acc = jnp.full((16,), -1, jnp.int32)
        for r in range(NWORK):
            acc = jnp.maximum(acc, offrows[pl.ds(r * OFFW + sb + kk * 16, 16)])
        offmer[pl.ds(kk * 16, 16)] = acc

    def put(ref, j, val):
        plsc.store_scatter(ref, [jnp.full((16,), j, jnp.int32)],
                           jnp.full((16,), val), mask=(iota == 0))

    def seg_step(j, cur_w):
        vv = offmer[pl.ds(j, 16)]
        s0 = jnp.maximum(vv[0], 0)
        e0 = jnp.maximum(vv[1], s0)

        def ensure(pos, cw):
            need = (pos < cw) | (pos + 16 > cw + WSIZE)
            nw = pl.multiple_of(jnp.where(need, pos - lax.rem(pos, 8), cw), 8)

            @pl.when(need)
            def _():
                pltpu.sync_copy(nl_hbm.at[pl.ds(nw, WSIZE)], nlbuf)
                pltpu.sync_copy(sc_hbm.at[pl.ds(nw, WSIZE)], scbuf)

            return nw

        def bodyA(carry):
            pos, cw, mvec, bvec, ivec, nlvec = carry
            cw = ensure(pos, cw)
            off = pos - cw
            v = nlbuf[pl.ds(off, 16)]
            sv = scbuf[pl.ds(off, 16)]
            lm = iota < (e0 - pos)
            vm = jnp.where(lm, v, NEG)
            svm = jnp.where(lm, sv, NEG)
            mvec = jnp.maximum(mvec, vm)
            take = svm >= bvec
            bvec = jnp.where(take, svm, bvec)
            ivec = jnp.where(take, pos + iota, ivec)
            nlvec = jnp.where(take, vm, nlvec)
            return pos + 16, cw, mvec, bvec, ivec, nlvec

        init = (s0, cur_w,
                jnp.full((16,), NEG), jnp.full((16,), NEG),
                jnp.full((16,), -1, jnp.int32), jnp.full((16,), NEG))
        pos, cur_w, mvec, bvec, ivec, nlvec = lax.while_loop(
            lambda c: c[0] < e0, bodyA, init)

        m = jnp.max(mvec)
        b2 = jnp.max(bvec)
        a = jnp.max(jnp.where(bvec == b2, ivec, -1))
        nla = jnp.max(jnp.where(ivec == a, nlvec, NEG))
        a = jnp.maximum(a, 0)

        def bodyB(carry):
            pos, cw, svec, dvec = carry
            cw = ensure(pos, cw)
            off = pos - cw
            v = nlbuf[pl.ds(off, 16)]
            lm = iota < (e0 - pos)
            sh = v - m
            ex = jnp.where(lm, jnp.exp(sh), jnp.float32(0.0))
            return pos + 16, cw, svec + ex, dvec + ex * sh

        initb = (s0, cur_w, jnp.zeros((16,), jnp.float32),
                 jnp.zeros((16,), jnp.float32))
        pos, cur_w, svec, dvec = lax.while_loop(
            lambda c: c[0] < e0, bodyB, initb)

        put(outm, j, m)
        put(outs, j, jnp.sum(svec))
        put(outd, j, jnp.sum(dvec))
        put(outnla, j, nla)
        put(outa, j, a)
        return cur_w

    lax.fori_loop(0, 32, seg_step, jnp.int32(-2 ** 30))

    pltpu.sync_copy(outm, m_hbm.at[pl.ds(sb, 32)])
    pltpu.sync_copy(outs, s_hbm.at[pl.ds(sb, 32)])
    pltpu.sync_copy(outd, d_hbm.at[pl.ds(sb, 32)])
    pltpu.sync_copy(outnla, nla_hbm.at[pl.ds(sb, 32)])
    pltpu.sync_copy(outa, a_hbm.at[pl.ds(sb, 32)])
    va = outa[pl.ds(0, 16)]
    vb = outa[pl.ds(16, 16)]
    rowscr[pl.ds(0, 16)] = lax.shift_right_logical(va, 3)
    rowscr[pl.ds(16, 16)] = lax.shift_right_logical(vb, 3)
    pltpu.async_copy(h_hbm.at[rowscr], rowsbuf, dsem).wait()
    for k in range(32):
        ak = va[k] if k < 16 else vb[k - 16]
        off = (ak & 7) * F
        hselloc[k // 8, pl.ds((k % 8) * F, F)] = rowsbuf[k, pl.ds(off, F)]
    for j in range(4):
        pltpu.sync_copy(hselloc.at[j],
                        hsel_hbm.at[pl.ds(sb * F + j * 128, 128)])


def _scb(nl, sc, h2d, offtabs):
    mesh = plsc.VectorSubcoreMesh(core_axis_name="c", subcore_axis_name="s")
    kern = pl.kernel(
        _scb_body,
        out_type=(
            jax.ShapeDtypeStruct((G,), jnp.float32),
            jax.ShapeDtypeStruct((G,), jnp.float32),
            jax.ShapeDtypeStruct((G,), jnp.float32),
            jax.ShapeDtypeStruct((G,), jnp.int32),
            jax.ShapeDtypeStruct((G,), jnp.float32),
            jax.ShapeDtypeStruct((G * F,), jnp.float32),
        ),
        mesh=mesh,
        scratch_types=[
            pltpu.VMEM((WSIZE,), jnp.float32),
            pltpu.VMEM((WSIZE,), jnp.float32),
            pltpu.VMEM((NWORK * OFFW,), jnp.int32),
            pltpu.VMEM((48,), jnp.int32),
            pltpu.VMEM((32,), jnp.float32),
            pltpu.VMEM((32,), jnp.float32),
            pltpu.VMEM((32,), jnp.float32),
            pltpu.VMEM((32,), jnp.float32),
            pltpu.VMEM((32,), jnp.int32),
            pltpu.VMEM((32,), jnp.int32),
            pltpu.VMEM((32, 128), jnp.float32),
            pltpu.VMEM((4, 128), jnp.float32),
            pltpu.SemaphoreType.DMA,
        ],
        compiler_params=pltpu.CompilerParams(needs_layout_passes=False),
    )
    return kern(nl, sc, h2d, offtabs)


def _tc2_body(hs, wa, ba, g2, mv, sv, dv, nla, lp_out, ent_out, act_out):
    al = jax.lax.dot_general(
        hs[...], wa[...], (((1,), (0,)), ((), ())),
        preferred_element_type=jnp.float32) + ba[...]
    rmax = jnp.max(al, axis=1, keepdims=True)
    sha = al - rmax
    lse = jnp.log(jnp.sum(jnp.exp(sha), axis=1, keepdims=True))
    logp = sha - lse
    sc2 = al + g2[...]
    smax = jnp.max(sc2, axis=1, keepdims=True)
    lanes = jax.lax.broadcasted_iota(jnp.int32, (G, N_ACT), 1)
    aidx = jnp.min(jnp.where(sc2 == smax, lanes, N_ACT), axis=1,
                   keepdims=True)
    alogp = jnp.sum(jnp.where(lanes == aidx, logp, 0.0), axis=1,
                    keepdims=True)
    aent = -jnp.sum(jnp.exp(logp) * logp, axis=1, keepdims=True)

    S = sv[...]
    logS = jnp.log(S)
    lp = (nla[...] - mv[...] - logS) + alogp.reshape(8, N_ACT)
    ent = (logS - dv[...] / S) + aent.reshape(8, N_ACT)
    lp_out[...] = lp
    ent_out[...] = ent
    act_out[...] = aidx.reshape(8, N_ACT)


def _tc2(hsel, wa, ba, g2, mv, sv, dv, nla):
    return pl.pallas_call(
        _tc2_body,
        out_shape=[
            jax.ShapeDtypeStruct((8, N_ACT), jnp.float32),
            jax.ShapeDtypeStruct((8, N_ACT), jnp.float32),
            jax.ShapeDtypeStruct((8, N_ACT), jnp.int32),
        ],
    )(hsel, wa, ba, g2, mv, sv, dv, nla)


def kernel(h, batch_idx, W_node, b_node, W_act, b_act):
    n = h.shape[0]

    kg = jax.random.key(42)
    g1 = jax.random.gumbel(kg, (n,), dtype=jnp.float32)
    g2 = jax.random.gumbel(jax.random.fold_in(kg, 1), (G, N_ACT),
                           dtype=jnp.float32)

    hlin = h.astype(jnp.float32).reshape(NROWS, 128)
    h3 = hlin.reshape(NROWS // RROW, RROW, 128)
    g13 = g1.reshape(NROWS // RROW, RROW, 8)
    wvec = W_node.astype(jnp.float32).reshape(1, F)
    mm = (jnp.tile(wvec, (8, 1)).reshape(128, 1)
          * jnp.repeat(jnp.eye(8, dtype=jnp.float32), F, axis=0))
    nl2, sc2 = _tc1(h3, g13, mm,
                    b_node.reshape(1, 1).astype(jnp.float32))
    nl = jnp.pad(nl2.reshape(n), (0, NPAD2 - n))
    sc = jnp.pad(sc2.reshape(n), (0, NPAD2 - n))

    offtabs = _sca(batch_idx.astype(jnp.int32), n)
    mv, sv, dv, nidx, nla, hselflat = _scb(nl, sc, hlin, offtabs)

    lp, ent, act = _tc2(
        hselflat.reshape(G, F), W_act.astype(jnp.float32),
        b_act.reshape(1, N_ACT).astype(jnp.float32),
        g2, mv.reshape(8, N_ACT), sv.reshape(8, N_ACT),
        dv.reshape(8, N_ACT), nla.reshape(8, N_ACT))

    actions = jnp.stack([nidx, act.reshape(G)], axis=-1)
    return actions, lp.reshape(G), ent.reshape(G)


# speedup vs baseline: 16.9979x; 1.1758x over previous
import functools
import jax
import jax.numpy as jnp
from jax import lax
from jax.experimental import pallas as pl
from jax.experimental.pallas import tpu as pltpu
from jax.experimental.pallas import tpu_sc as plsc

N_ACT = 128
G = 1024
F = 16
NWORK = 32
RPW = 3136
NPAD = NWORK * RPW
WSIZE = 8192
NPAD2 = NPAD + WSIZE
OFFW = 1040
BL = 2048
NEG = -1e9


RROW = 500
NROWS = 12500


def _tc1_body(hb, mm, bn, nl_out):
    pk = jax.lax.dot_general(
        hb[0], mm[...], (((1,), (0,)), ((), ())),
        preferred_element_type=jnp.float32) + bn[0, 0]
    nl_out[0] = pk


def _tc1(h3, mm, bn):
    nb = NROWS // RROW
    return pl.pallas_call(
        _tc1_body,
        grid=(nb,),
        in_specs=[
            pl.BlockSpec((1, RROW, 128), lambda j: (j, 0, 0)),
            pl.BlockSpec((128, 8), lambda j: (0, 0)),
            pl.BlockSpec((1, 1), lambda j: (0, 0)),
        ],
        out_specs=pl.BlockSpec((1, RROW, 8), lambda j: (j, 0, 0)),
        out_shape=jax.ShapeDtypeStruct((nb, RROW, 8), jnp.float32),
    )(h3, mm, bn)


def _sca_body(n_real, bidx_hbm, offtab_hbm, bbuf, offloc, dsem):
    widx = lax.axis_index("s") * 2 + lax.axis_index("c")
    iota = lax.iota(jnp.int32, 16)
    neg1 = jnp.full((16,), -1, jnp.int32)
    last_n = n_real - (NWORK - 1) * RPW

    @pl.when(widx < NWORK - 1)
    def _():
        pltpu.sync_copy(bidx_hbm.at[pl.ds(widx * RPW, RPW)],
                        bbuf.at[pl.ds(16, RPW)])

    @pl.when(widx == NWORK - 1)
    def _():
        pltpu.sync_copy(bidx_hbm.at[pl.ds((NWORK - 1) * RPW, last_n)],
                        bbuf.at[pl.ds(16, last_n)])

    @pl.when(widx > 0)
    def _():
        pltpu.sync_copy(bidx_hbm.at[pl.ds(widx * RPW - 16, 16)],
                        bbuf.at[pl.ds(0, 16)])

    @pl.when(widx == 0)
    def _():
        bbuf[pl.ds(0, 16)] = neg1

    for i in range(OFFW // 16):
        offloc[pl.ds(i * 16, 16)] = neg1
    offloc[pl.ds(G, 16)] = jnp.where(iota == 0, n_real, -1)

    def step(c, _):
        cur = bbuf[pl.ds(16 + c * 16, 16)]
        prev = bbuf[pl.ds(15 + c * 16, 16)]
        bmask = cur != prev
        posv = widx * RPW + c * 16 + iota
        plsc.store_scatter(offloc, [cur], posv, mask=bmask)
        return _

    nchunks = jnp.where(widx == NWORK - 1, last_n // 16, RPW // 16)
    lax.fori_loop(0, nchunks, step, 0)
    pltpu.sync_copy(offloc, offtab_hbm.at[pl.ds(widx * OFFW, OFFW)])


def _sca(bidx_p, n_real):
    mesh = plsc.VectorSubcoreMesh(core_axis_name="c", subcore_axis_name="s")
    kern = pl.kernel(
        functools.partial(_sca_body, n_real),
        out_type=jax.ShapeDtypeStruct((NWORK * OFFW,), jnp.int32),
        mesh=mesh,
        scratch_types=[
            pltpu.VMEM((RPW + 16,), jnp.int32),
            pltpu.VMEM((OFFW,), jnp.int32),
            pltpu.SemaphoreType.DMA,
        ],
        compiler_params=pltpu.CompilerParams(needs_layout_passes=False),
    )
    return kern(bidx_p)


def _scb_body(nl_hbm, sc_hbm, h_hbm, offtab_hbm,
              m_hbm, s_hbm, d_hbm, a_hbm, nla_hbm, hsel_hbm,
              nlbuf, scbuf, offrows, offmer,
              outm, outs, outd, outnla, outa, rowscr, rowsbuf,
              hselloc, dsem):
    widx = lax.axis_index("s") * 2 + lax.axis_index("c")
    iota = lax.iota(jnp.int32, 16)
    sb = widx * 32

    pltpu.sync_copy(offtab_hbm, offrows)
    for kk in range(3):
        acc = jnp.full((16,), -1, jnp.int32)
        for r in range(NWORK):
            acc = jnp.maximum(acc, offrows[pl.ds(r * OFFW + sb + kk * 16, 16)])
        offmer[pl.ds(kk * 16, 16)] = acc

    def put(ref, j, val):
        plsc.store_scatter(ref, [jnp.full((16,), j, jnp.int32)],
                           jnp.full((16,), val), mask=(iota == 0))

    def seg_step(j, cur_w):
        vv = offmer[pl.ds(j, 16)]
        s0 = jnp.maximum(vv[0], 0)
        e0 = jnp.maximum(vv[1], s0)

        def ensure(pos, cw):
            need = (pos < cw) | (pos + 16 > cw + WSIZE)
            nw = pl.multiple_of(jnp.where(need, pos - lax.rem(pos, 8), cw), 8)

            @pl.when(need)
            def _():
                pltpu.sync_copy(nl_hbm.at[pl.ds(nw, WSIZE)], nlbuf)
                pltpu.sync_copy(sc_hbm.at[pl.ds(nw, WSIZE)], scbuf)

            return nw

        def bodyA(carry):
            pos, cw, mvec, bvec, ivec, nlvec = carry
            cw = ensure(pos, cw)
            off = pos - cw
            v = nlbuf[pl.ds(off, 16)]
            sv = v + scbuf[pl.ds(off, 16)]
            lm = iota < (e0 - pos)
            vm = jnp.where(lm, v, NEG)
            svm = jnp.where(lm, sv, NEG)
            mvec = jnp.maximum(mvec, vm)
            take = svm >= bvec
            bvec = jnp.where(take, svm, bvec)
            ivec = jnp.where(take, pos + iota, ivec)
            nlvec = jnp.where(take, vm, nlvec)
            return pos + 16, cw, mvec, bvec, ivec, nlvec

        init = (s0, cur_w,
                jnp.full((16,), NEG), jnp.full((16,), NEG),
                jnp.full((16,), -1, jnp.int32), jnp.full((16,), NEG))
        pos, cur_w, mvec, bvec, ivec, nlvec = lax.while_loop(
            lambda c: c[0] < e0, bodyA, init)

        m = jnp.max(mvec)
        b2 = jnp.max(bvec)
        a = jnp.max(jnp.where(bvec == b2, ivec, -1))
        nla = jnp.max(jnp.where(ivec == a, nlvec, NEG))
        a = jnp.maximum(a, 0)

        def bodyB(carry):
            pos, cw, svec, dvec = carry
            cw = ensure(pos, cw)
            off = pos - cw
            v = nlbuf[pl.ds(off, 16)]
            lm = iota < (e0 - pos)
            sh = v - m
            ex = jnp.where(lm, jnp.exp(sh), jnp.float32(0.0))
            return pos + 16, cw, svec + ex, dvec + ex * sh

        initb = (s0, cur_w, jnp.zeros((16,), jnp.float32),
                 jnp.zeros((16,), jnp.float32))
        pos, cur_w, svec, dvec = lax.while_loop(
            lambda c: c[0] < e0, bodyB, initb)

        put(outm, j, m)
        put(outs, j, jnp.sum(svec))
        put(outd, j, jnp.sum(dvec))
        put(outnla, j, nla)
        put(outa, j, a)
        return cur_w

    lax.fori_loop(0, 32, seg_step, jnp.int32(-2 ** 30))

    pltpu.sync_copy(outm, m_hbm.at[pl.ds(sb, 32)])
    pltpu.sync_copy(outs, s_hbm.at[pl.ds(sb, 32)])
    pltpu.sync_copy(outd, d_hbm.at[pl.ds(sb, 32)])
    pltpu.sync_copy(outnla, nla_hbm.at[pl.ds(sb, 32)])
    pltpu.sync_copy(outa, a_hbm.at[pl.ds(sb, 32)])
    va = outa[pl.ds(0, 16)]
    vb = outa[pl.ds(16, 16)]
    rowscr[pl.ds(0, 16)] = lax.shift_right_logical(va, 3)
    rowscr[pl.ds(16, 16)] = lax.shift_right_logical(vb, 3)
    pltpu.async_copy(h_hbm.at[rowscr], rowsbuf, dsem).wait()
    for k in range(32):
        ak = va[k] if k < 16 else vb[k - 16]
        off = (ak & 7) * F
        hselloc[k // 8, pl.ds((k % 8) * F, F)] = rowsbuf[k, pl.ds(off, F)]
    for j in range(4):
        pltpu.sync_copy(hselloc.at[j],
                        hsel_hbm.at[pl.ds(sb * F + j * 128, 128)])


def _scb(nl, sc, h2d, offtabs):
    mesh = plsc.VectorSubcoreMesh(core_axis_name="c", subcore_axis_name="s")
    kern = pl.kernel(
        _scb_body,
        out_type=(
            jax.ShapeDtypeStruct((G,), jnp.float32),
            jax.ShapeDtypeStruct((G,), jnp.float32),
            jax.ShapeDtypeStruct((G,), jnp.float32),
            jax.ShapeDtypeStruct((G,), jnp.int32),
            jax.ShapeDtypeStruct((G,), jnp.float32),
            jax.ShapeDtypeStruct((G * F,), jnp.float32),
        ),
        mesh=mesh,
        scratch_types=[
            pltpu.VMEM((WSIZE,), jnp.float32),
            pltpu.VMEM((WSIZE,), jnp.float32),
            pltpu.VMEM((NWORK * OFFW,), jnp.int32),
            pltpu.VMEM((48,), jnp.int32),
            pltpu.VMEM((32,), jnp.float32),
            pltpu.VMEM((32,), jnp.float32),
            pltpu.VMEM((32,), jnp.float32),
            pltpu.VMEM((32,), jnp.float32),
            pltpu.VMEM((32,), jnp.int32),
            pltpu.VMEM((32,), jnp.int32),
            pltpu.VMEM((32, 128), jnp.float32),
            pltpu.VMEM((4, 128), jnp.float32),
            pltpu.SemaphoreType.DMA,
        ],
        compiler_params=pltpu.CompilerParams(needs_layout_passes=False),
    )
    return kern(nl, sc, h2d, offtabs)


def _tc2_body(hs, wa, ba, g2, mv, sv, dv, nla, lp_out, ent_out, act_out):
    al = jax.lax.dot_general(
        hs[...], wa[...], (((1,), (0,)), ((), ())),
        preferred_element_type=jnp.float32) + ba[...]
    rmax = jnp.max(al, axis=1, keepdims=True)
    sha = al - rmax
    lse = jnp.log(jnp.sum(jnp.exp(sha), axis=1, keepdims=True))
    logp = sha - lse
    sc2 = al + g2[...]
    smax = jnp.max(sc2, axis=1, keepdims=True)
    lanes = jax.lax.broadcasted_iota(jnp.int32, (G, N_ACT), 1)
    aidx = jnp.min(jnp.where(sc2 == smax, lanes, N_ACT), axis=1,
                   keepdims=True)
    alogp = jnp.sum(jnp.where(lanes == aidx, logp, 0.0), axis=1,
                    keepdims=True)
    aent = -jnp.sum(jnp.exp(logp) * logp, axis=1, keepdims=True)

    S = sv[...]
    logS = jnp.log(S)
    lp = (nla[...] - mv[...] - logS) + alogp.reshape(8, N_ACT)
    ent = (logS - dv[...] / S) + aent.reshape(8, N_ACT)
    lp_out[...] = lp
    ent_out[...] = ent
    act_out[...] = aidx.reshape(8, N_ACT)


def _tc2(hsel, wa, ba, g2, mv, sv, dv, nla):
    return pl.pallas_call(
        _tc2_body,
        out_shape=[
            jax.ShapeDtypeStruct((8, N_ACT), jnp.float32),
            jax.ShapeDtypeStruct((8, N_ACT), jnp.float32),
            jax.ShapeDtypeStruct((8, N_ACT), jnp.int32),
        ],
    )(hsel, wa, ba, g2, mv, sv, dv, nla)


def kernel(h, batch_idx, W_node, b_node, W_act, b_act):
    n = h.shape[0]

    kg = jax.random.key(42)
    g1 = jax.random.gumbel(kg, (n,), dtype=jnp.float32)
    g2 = jax.random.gumbel(jax.random.fold_in(kg, 1), (G, N_ACT),
                           dtype=jnp.float32)

    hlin = h.astype(jnp.float32).reshape(NROWS, 128)
    h3 = hlin.reshape(NROWS // RROW, RROW, 128)
    wvec = W_node.astype(jnp.float32).reshape(1, F)
    mm = (jnp.tile(wvec, (8, 1)).reshape(128, 1)
          * jnp.repeat(jnp.eye(8, dtype=jnp.float32), F, axis=0))
    nl2 = _tc1(h3, mm, b_node.reshape(1, 1).astype(jnp.float32))
    nl = jnp.pad(nl2.reshape(n), (0, NPAD2 - n))
    g1p = jnp.pad(g1, (0, NPAD2 - n))

    offtabs = _sca(batch_idx.astype(jnp.int32), n)
    mv, sv, dv, nidx, nla, hselflat = _scb(nl, g1p, hlin, offtabs)

    lp, ent, act = _tc2(
        hselflat.reshape(G, F), W_act.astype(jnp.float32),
        b_act.reshape(1, N_ACT).astype(jnp.float32),
        g2, mv.reshape(8, N_ACT), sv.reshape(8, N_ACT),
        dv.reshape(8, N_ACT), nla.reshape(8, N_ACT))

    actions = jnp.stack([nidx, act.reshape(G)], axis=-1)
    return actions, lp.reshape(G), ent.reshape(G)


# speedup vs baseline: 17.0044x; 1.0004x over previous
import functools
import jax
import jax.numpy as jnp
from jax import lax
from jax.experimental import pallas as pl
from jax.experimental.pallas import tpu as pltpu
from jax.experimental.pallas import tpu_sc as plsc

N_ACT = 128
G = 1024
F = 16
NWORK = 32
RPW = 3136
NPAD = NWORK * RPW
WSIZE = 8192
NPAD2 = NPAD + WSIZE
OFFW = 1040
NEG = -1e9


RROW = 500
NROWS = 12500


def _tc1_body(hb, mm, bn, nl_out):
    pk = jax.lax.dot_general(
        hb[0], mm[...], (((1,), (0,)), ((), ())),
        preferred_element_type=jnp.float32) + bn[0, 0]
    nl_out[0] = pk


def _tc1(h3, mm, bn):
    nb = NROWS // RROW
    return pl.pallas_call(
        _tc1_body,
        grid=(nb,),
        in_specs=[
            pl.BlockSpec((1, RROW, 128), lambda j: (j, 0, 0)),
            pl.BlockSpec((128, 8), lambda j: (0, 0)),
            pl.BlockSpec((1, 1), lambda j: (0, 0)),
        ],
        out_specs=pl.BlockSpec((1, RROW, 8), lambda j: (j, 0, 0)),
        out_shape=jax.ShapeDtypeStruct((nb, RROW, 8), jnp.float32),
    )(h3, mm, bn)


def _sca_body(n_real, bidx_hbm, offtab_hbm, bbuf, offloc, dsem):
    widx = lax.axis_index("s") * 2 + lax.axis_index("c")
    iota = lax.iota(jnp.int32, 16)
    neg1 = jnp.full((16,), -1, jnp.int32)
    last_n = n_real - (NWORK - 1) * RPW

    @pl.when(widx < NWORK - 1)
    def _():
        pltpu.sync_copy(bidx_hbm.at[pl.ds(widx * RPW, RPW)],
                        bbuf.at[pl.ds(16, RPW)])

    @pl.when(widx == NWORK - 1)
    def _():
        pltpu.sync_copy(bidx_hbm.at[pl.ds((NWORK - 1) * RPW, last_n)],
                        bbuf.at[pl.ds(16, last_n)])

    @pl.when(widx > 0)
    def _():
        pltpu.sync_copy(bidx_hbm.at[pl.ds(widx * RPW - 16, 16)],
                        bbuf.at[pl.ds(0, 16)])

    @pl.when(widx == 0)
    def _():
        bbuf[pl.ds(0, 16)] = neg1

    for i in range(OFFW // 16):
        offloc[pl.ds(i * 16, 16)] = neg1
    offloc[pl.ds(G, 16)] = jnp.where(iota == 0, n_real, -1)

    def step(c, _):
        cur = bbuf[pl.ds(16 + c * 16, 16)]
        prev = bbuf[pl.ds(15 + c * 16, 16)]
        bmask = cur != prev
        posv = widx * RPW + c * 16 + iota
        plsc.store_scatter(offloc, [cur], posv, mask=bmask)
        return _

    nchunks = jnp.where(widx == NWORK - 1, last_n // 16, RPW // 16)
    lax.fori_loop(0, nchunks, step, 0)
    pltpu.sync_copy(offloc, offtab_hbm.at[pl.ds(widx * OFFW, OFFW)])


def _sca(bidx_p, n_real):
    mesh = plsc.VectorSubcoreMesh(core_axis_name="c", subcore_axis_name="s")
    kern = pl.kernel(
        functools.partial(_sca_body, n_real),
        out_type=jax.ShapeDtypeStruct((NWORK * OFFW,), jnp.int32),
        mesh=mesh,
        scratch_types=[
            pltpu.VMEM((RPW + 16,), jnp.int32),
            pltpu.VMEM((OFFW,), jnp.int32),
            pltpu.SemaphoreType.DMA,
        ],
        compiler_params=pltpu.CompilerParams(needs_layout_passes=False),
    )
    return kern(bidx_p)


def _scb_body(nl_hbm, sc_hbm, h_hbm, offtab_hbm,
              m_hbm, s_hbm, d_hbm, a_hbm, nla_hbm, hsel_hbm,
              nlbuf, scbuf, offrows, offmer,
              outm, outs, outd, outnla, outa, rowscr, rowsbuf,
              hselloc, dsem):
    widx = lax.axis_index("s") * 2 + lax.axis_index("c")
    iota = lax.iota(jnp.int32, 16)
    sb = widx * 32

    pltpu.sync_copy(offtab_hbm, offrows)
    for kk in range(3):
        acc = jnp.full((16,), -1, jnp.int32)
        for r in range(NWORK):
            acc = jnp.maximum(acc, offrows[pl.ds(r * OFFW + sb + kk * 16, 16)])
        offmer[pl.ds(kk * 16, 16)] = acc

    def put(ref, j, val):
        plsc.store_scatter(ref, [jnp.full((16,), j, jnp.int32)],
                           jnp.full((16,), val), mask=(iota == 0))

    def seg_step(j, cur_w):
        vv = offmer[pl.ds(j, 16)]
        s0 = jnp.maximum(vv[0], 0)
        e0 = jnp.maximum(vv[1], s0)

        def ensure(pos, cw):
            need = (pos < cw) | (pos + 16 > cw + WSIZE)
            nw = pl.multiple_of(jnp.where(need, pos - lax.rem(pos, 8), cw), 8)

            @pl.when(need)
            def _():
                pltpu.sync_copy(nl_hbm.at[pl.ds(nw, WSIZE)], nlbuf)
                pltpu.sync_copy(sc_hbm.at[pl.ds(nw, WSIZE)], scbuf)

            return nw

        def bodyA(carry):
            pos, cw, mvec, bvec, ivec, nlvec = carry
            cw = ensure(pos, cw)
            off = pos - cw
            v = nlbuf[pl.ds(off, 16)]
            sv = v + scbuf[pl.ds(off, 16)]
            lm = iota < (e0 - pos)
            vm = jnp.where(lm, v, NEG)
            svm = jnp.where(lm, sv, NEG)
            mvec = jnp.maximum(mvec, vm)
            take = svm >= bvec
            bvec = jnp.where(take, svm, bvec)
            ivec = jnp.where(take, pos + iota, ivec)
            nlvec = jnp.where(take, vm, nlvec)
            return pos + 16, cw, mvec, bvec, ivec, nlvec

        init = (s0, cur_w,
                jnp.full((16,), NEG), jnp.full((16,), NEG),
                jnp.full((16,), -1, jnp.int32), jnp.full((16,), NEG))
        pos, cur_w, mvec, bvec, ivec, nlvec = lax.while_loop(
            lambda c: c[0] < e0, bodyA, init)

        m = jnp.max(mvec)
        b2 = jnp.max(bvec)
        a = jnp.max(jnp.where(bvec == b2, ivec, -1))
        nla = jnp.max(jnp.where(ivec == a, nlvec, NEG))
        a = jnp.maximum(a, 0)

        def bodyB(carry):
            pos, cw, svec, dvec = carry
            cw = ensure(pos, cw)
            off = pos - cw
            v = nlbuf[pl.ds(off, 16)]
            lm = iota < (e0 - pos)
            sh = v - m
            ex = jnp.where(lm, jnp.exp(sh), jnp.float32(0.0))
            return pos + 16, cw, svec + ex, dvec + ex * sh

        initb = (s0, cur_w, jnp.zeros((16,), jnp.float32),
                 jnp.zeros((16,), jnp.float32))
        pos, cur_w, svec, dvec = lax.while_loop(
            lambda c: c[0] < e0, bodyB, initb)

        put(outm, j, m)
        put(outs, j, jnp.sum(svec))
        put(outd, j, jnp.sum(dvec))
        put(outnla, j, nla)
        put(outa, j, a)
        return cur_w

    lax.fori_loop(0, 32, seg_step, jnp.int32(-2 ** 30))

    pltpu.sync_copy(outm, m_hbm.at[pl.ds(sb, 32)])
    pltpu.sync_copy(outs, s_hbm.at[pl.ds(sb, 32)])
    pltpu.sync_copy(outd, d_hbm.at[pl.ds(sb, 32)])
    pltpu.sync_copy(outnla, nla_hbm.at[pl.ds(sb, 32)])
    pltpu.sync_copy(outa, a_hbm.at[pl.ds(sb, 32)])
    va = outa[pl.ds(0, 16)]
    vb = outa[pl.ds(16, 16)]
    rowscr[pl.ds(0, 16)] = lax.shift_right_logical(va, 3)
    rowscr[pl.ds(16, 16)] = lax.shift_right_logical(vb, 3)
    pltpu.async_copy(h_hbm.at[rowscr], rowsbuf, dsem).wait()
    for k in range(32):
        ak = va[k] if k < 16 else vb[k - 16]
        off = (ak & 7) * F
        hselloc[k // 8, pl.ds((k % 8) * F, F)] = rowsbuf[k, pl.ds(off, F)]
    for j in range(4):
        pltpu.sync_copy(hselloc.at[j],
                        hsel_hbm.at[pl.ds(sb * F + j * 128, 128)])


def _scb(nl, sc, h2d, offtabs):
    mesh = plsc.VectorSubcoreMesh(core_axis_name="c", subcore_axis_name="s")
    kern = pl.kernel(
        _scb_body,
        out_type=(
            jax.ShapeDtypeStruct((G,), jnp.float32),
            jax.ShapeDtypeStruct((G,), jnp.float32),
            jax.ShapeDtypeStruct((G,), jnp.float32),
            jax.ShapeDtypeStruct((G,), jnp.int32),
            jax.ShapeDtypeStruct((G,), jnp.float32),
            jax.ShapeDtypeStruct((G * F,), jnp.float32),
        ),
        mesh=mesh,
        scratch_types=[
            pltpu.VMEM((WSIZE,), jnp.float32),
            pltpu.VMEM((WSIZE,), jnp.float32),
            pltpu.VMEM((NWORK * OFFW,), jnp.int32),
            pltpu.VMEM((48,), jnp.int32),
            pltpu.VMEM((32,), jnp.float32),
            pltpu.VMEM((32,), jnp.float32),
            pltpu.VMEM((32,), jnp.float32),
            pltpu.VMEM((32,), jnp.float32),
            pltpu.VMEM((32,), jnp.int32),
            pltpu.VMEM((32,), jnp.int32),
            pltpu.VMEM((32, 128), jnp.float32),
            pltpu.VMEM((4, 128), jnp.float32),
            pltpu.SemaphoreType.DMA,
        ],
        compiler_params=pltpu.CompilerParams(needs_layout_passes=False),
    )
    return kern(nl, sc, h2d, offtabs)


def _tc2_body(hs, wa, ba, g2, mv, sv, dv, nla, lp_out, ent_out, act_out):
    al = jax.lax.dot_general(
        hs[...], wa[...], (((1,), (0,)), ((), ())),
        preferred_element_type=jnp.float32) + ba[...]
    rmax = jnp.max(al, axis=1, keepdims=True)
    sha = al - rmax
    lse = jnp.log(jnp.sum(jnp.exp(sha), axis=1, keepdims=True))
    logp = sha - lse
    sc2 = al + g2[...]
    smax = jnp.max(sc2, axis=1, keepdims=True)
    lanes = jax.lax.broadcasted_iota(jnp.int32, (G, N_ACT), 1)
    aidx = jnp.min(jnp.where(sc2 == smax, lanes, N_ACT), axis=1,
                   keepdims=True)
    alogp = jnp.sum(jnp.where(lanes == aidx, logp, 0.0), axis=1,
                    keepdims=True)
    aent = -jnp.sum(jnp.exp(logp) * logp, axis=1, keepdims=True)

    S = sv[...]
    logS = jnp.log(S)
    lp = (nla[...] - mv[...] - logS) + alogp.reshape(8, N_ACT)
    ent = (logS - dv[...] / S) + aent.reshape(8, N_ACT)
    lp_out[...] = lp
    ent_out[...] = ent
    act_out[...] = aidx.reshape(8, N_ACT)


def _tc2(hsel, wa, ba, g2, mv, sv, dv, nla):
    return pl.pallas_call(
        _tc2_body,
        out_shape=[
            jax.ShapeDtypeStruct((8, N_ACT), jnp.float32),
            jax.ShapeDtypeStruct((8, N_ACT), jnp.float32),
            jax.ShapeDtypeStruct((8, N_ACT), jnp.int32),
        ],
    )(hsel, wa, ba, g2, mv, sv, dv, nla)


def kernel(h, batch_idx, W_node, b_node, W_act, b_act):
    n = h.shape[0]

    kg = jax.random.key(42)
    g1 = jax.random.gumbel(kg, (n,), dtype=jnp.float32)
    g2 = jax.random.gumbel(jax.random.fold_in(kg, 1), (G, N_ACT),
                           dtype=jnp.float32)

    hlin = h.astype(jnp.float32).reshape(NROWS, 128)
    h3 = hlin.reshape(NROWS // RROW, RROW, 128)
    wvec = W_node.astype(jnp.float32).reshape(1, F)
    mm = (jnp.tile(wvec, (8, 1)).reshape(128, 1)
          * jnp.repeat(jnp.eye(8, dtype=jnp.float32), F, axis=0))
    nl2 = _tc1(h3, mm, b_node.reshape(1, 1).astype(jnp.float32))
    nl = jnp.pad(nl2.reshape(n), (0, NPAD2 - n))
    g1p = jnp.pad(g1, (0, NPAD2 - n))

    offtabs = _sca(batch_idx.astype(jnp.int32), n)
    mv, sv, dv, nidx, nla, hselflat = _scb(nl, g1p, hlin, offtabs)

    lp, ent, act = _tc2(
        hselflat.reshape(G, F), W_act.astype(jnp.float32),
        b_act.reshape(1, N_ACT).astype(jnp.float32),
        g2, mv.reshape(8, N_ACT), sv.reshape(8, N_ACT),
        dv.reshape(8, N_ACT), nla.reshape(8, N_ACT))

    actions = jnp.stack([nidx, act.reshape(G)], axis=-1)
    return actions, lp.reshape(G), ent.reshape(G)
